# baseline (speedup 1.0000x reference)
import functools

import jax
import jax.numpy as jnp
from jax import lax
from jax.experimental import pallas as pl
from jax.experimental.pallas import tpu as pltpu
from jax.experimental.pallas import tpu_sc as plsc

BATCH = 16384
N = 20
D = 16
W = 48
K = 6
NC = 2
NS = 16
NW = NC * NS
BPW = BATCH // NW
C = 32
NCH = BPW // C
G = 128
NG = C * N // G


def _branch_accum(e20, comb, w1_v, br, acc):
    erows = [comb[e20 + j, pl.ds(0, 16)] for j in range(N)]
    for i in range(N):
        rvec = jnp.full((16,), e20 + i, jnp.int32)
        parts = [None, None, None, None]
        for j in range(N):
            svec = plsc.load_gather(
                comb, [rvec, jnp.full((16,), D + j, jnp.int32)])
            sij = svec * erows[j]
            p = j & 3
            parts[p] = sij if parts[p] is None else parts[p] + sij
        scored = (parts[0] + parts[1]) + (parts[2] + parts[3])
        for k in range(K):
            acc[k] = acc[k] + scored * w1_v[br, k, i, :]
    return acc


def _flatten_rows(src2d, dst2d):
    lanes = lax.iota(jnp.int32, 16)
    for t in range(C * N // 16):
        r0, rem = divmod(t * 16, N)
        wrap = (rem + lanes) >= N
        r = jnp.where(wrap, r0 + 1, r0)
        col = jnp.where(wrap, rem + lanes - N, rem + lanes)
        v = plsc.load_gather(src2d, [r, col])
        g, o = divmod(t * 16, G)
        dst2d[g, pl.ds(o, 16)] = v


def _sc_body(uidx_hbm, iidx_hbm, uit_hbm, iit_hbm, uc_hbm, ic_hbm,
             w1_hbm, par_hbm, out_hbm,
             uc_v, ic_v, un_v, in_v, unf_v, inf_v, ucb_v, icb_v,
             w1_v, par_v, hbuf_v, out_v, sem0, sem1, sem2, sem3):
    wid = lax.axis_index("s") * NC + lax.axis_index("c")
    base = wid * BPW

    pltpu.sync_copy(w1_hbm, w1_v)
    pltpu.sync_copy(par_hbm, par_v)

    lanes = lax.iota(jnp.int32, 16)
    lane0 = lanes == 0
    b1vec = par_v[0, :]
    w2vec = par_v[1, :]
    b2s = w2vec[15]
    w2z = jnp.where(lanes < K, w2vec, 0.0)

    def chunk_body(c, _):
        cb = c * C
        cp0 = pltpu.async_copy(uidx_hbm.at[pl.ds(base + cb, C)], uc_v, sem0)
        cp1 = pltpu.async_copy(iidx_hbm.at[pl.ds(base + cb, C)], ic_v, sem1)
        cp0.wait()
        cp1.wait()
        cp0 = pltpu.async_copy(uit_hbm.at[uc_v], un_v, sem0)
        cp1 = pltpu.async_copy(iit_hbm.at[ic_v], in_v, sem1)
        cp0.wait()
        cp1.wait()
        _flatten_rows(un_v, unf_v)
        _flatten_rows(in_v, inf_v)
        copies = []
        for g in range(NG):
            s = pl.ds(g * G, G)
            copies.append(pltpu.async_copy(uc_hbm.at[unf_v.at[g]], ucb_v.at[s], sem0))
            copies.append(pltpu.async_copy(ic_hbm.at[inf_v.at[g]], icb_v.at[s], sem1))
        for cp in copies:
            cp.wait()

        def elem_body(e, _):
            e20 = e * N
            acc = [jnp.zeros((16,), jnp.float32) for _ in range(K)]
            acc = _branch_accum(e20, ucb_v, w1_v, 0, acc)
            acc = _branch_accum(e20, icb_v, w1_v, 1, acc)
            hvec = jnp.zeros((16,), jnp.float32)
            for k in range(K):
                hvec = hvec + jnp.where(lanes == k, jnp.sum(acc[k]), 0.0)
            hvec = jnp.maximum(hvec + b1vec, 0.0)
            z = jnp.sum(hvec * w2z) + b2s
            plsc.store_scatter(hbuf_v, [jnp.full((16,), cb + e, jnp.int32)],
                               jnp.full((16,), z, jnp.float32), mask=lane0)
            return 0

        lax.fori_loop(0, C, elem_body, 0)
        return 0

    lax.fori_loop(0, NCH, chunk_body, 0)

    for g in range(BPW // 16):
        z = hbuf_v[pl.ds(g * 16, 16)]
        z = jnp.maximum(z, 0.0)
        out_v[pl.ds(g * 16, 16)] = 1.0 / (1.0 + jnp.exp(-z))

    pltpu.sync_copy(out_v, out_hbm.at[pl.ds(base, BPW)])


@functools.partial(jax.jit, static_argnames=())
def kernel(user_idxs, item_idxs, user_idx_tensor, item_idx_tensor,
           user_scr_tensor, item_scr_tensor, user_emb_table, item_emb_table,
           W1, b1, W2, b2):
    padcols = jnp.zeros((user_emb_table.shape[0], W - D - N), jnp.float32)
    ucomb = jnp.concatenate([user_emb_table, user_scr_tensor, padcols], axis=1)
    icomb = jnp.concatenate([item_emb_table, item_scr_tensor, padcols], axis=1)
    w1r = W1.reshape(2, N, D, K).transpose(0, 3, 1, 2)
    w1r = jnp.asarray(w1r, jnp.float32)
    par = jnp.zeros((2, 16), jnp.float32)
    par = par.at[0, 0:K].set(b1)
    par = par.at[1, 0:K].set(W2[:, 0])
    par = par.at[1, 15].set(b2[0])

    run = pl.kernel(
        _sc_body,
        out_type=jax.ShapeDtypeStruct((BATCH,), jnp.float32),
        mesh=plsc.VectorSubcoreMesh(core_axis_name="c", subcore_axis_name="s"),
        compiler_params=pltpu.CompilerParams(
            needs_layout_passes=False, use_tc_tiling_on_sc=False),
        scratch_types=[
            pltpu.VMEM((C,), jnp.int32),
            pltpu.VMEM((C,), jnp.int32),
            pltpu.VMEM((C, N), jnp.int32),
            pltpu.VMEM((C, N), jnp.int32),
            pltpu.VMEM((NG, G), jnp.int32),
            pltpu.VMEM((NG, G), jnp.int32),
            pltpu.VMEM((C * N, W), jnp.float32),
            pltpu.VMEM((C * N, W), jnp.float32),
            pltpu.VMEM((2, K, N, D), jnp.float32),
            pltpu.VMEM((2, 16), jnp.float32),
            pltpu.VMEM((BPW,), jnp.float32),
            pltpu.VMEM((BPW,), jnp.float32),
            pltpu.SemaphoreType.DMA,
            pltpu.SemaphoreType.DMA,
            pltpu.SemaphoreType.DMA,
            pltpu.SemaphoreType.DMA,
        ],
    )
    return run(user_idxs, item_idxs, user_idx_tensor, item_idx_tensor,
               ucomb, icomb, w1r, par)


# speedup vs baseline: 2.2180x; 2.2180x over previous
import functools

import jax
import jax.numpy as jnp
from jax import lax
from jax.experimental import pallas as pl
from jax.experimental.pallas import tpu as pltpu
from jax.experimental.pallas import tpu_sc as plsc

BATCH = 16384
N = 20
D = 16
W = 48
K = 6
NC = 2
NS = 16
NW = NC * NS
BPW = BATCH // NW
C = 32
NCH = BPW // C
G = 128
NG = C * N // G


def _branch_accum(e20, comb, w1_v, br, acc):
    erows = [comb[e20 + j, pl.ds(0, 16)] for j in range(N)]
    for i in range(N):
        srow_lo = comb[e20 + i, pl.ds(D, 16)]
        srow_hi = comb[e20 + i, pl.ds(D + 16, 16)]
        parts = [None, None, None, None]
        for j in range(N):
            s = srow_lo[j] if j < 16 else srow_hi[j - 16]
            sij = s * erows[j]
            p = j & 3
            parts[p] = sij if parts[p] is None else parts[p] + sij
        scored = (parts[0] + parts[1]) + (parts[2] + parts[3])
        for k in range(K):
            acc[k] = acc[k] + scored * w1_v[br, k, i, :]
    return acc


def _flatten_rows(src2d, dst2d):
    lanes = lax.iota(jnp.int32, 16)
    for t in range(C * N // 16):
        r0, rem = divmod(t * 16, N)
        wrap = (rem + lanes) >= N
        r = jnp.where(wrap, r0 + 1, r0)
        col = jnp.where(wrap, rem + lanes - N, rem + lanes)
        v = plsc.load_gather(src2d, [r, col])
        g, o = divmod(t * 16, G)
        dst2d[g, pl.ds(o, 16)] = v


def _sc_body(uidx_hbm, iidx_hbm, uit_hbm, iit_hbm, uc_hbm, ic_hbm,
             w1_hbm, par_hbm, out_hbm,
             uc_v, ic_v, un_v, in_v, unf_v, inf_v, ucb_v, icb_v,
             w1_v, par_v, hbuf_v, out_v, sem0, sem1, sem2, sem3):
    wid = lax.axis_index("s") * NC + lax.axis_index("c")
    base = wid * BPW

    pltpu.sync_copy(w1_hbm, w1_v)
    pltpu.sync_copy(par_hbm, par_v)

    lanes = lax.iota(jnp.int32, 16)
    lane0 = lanes == 0
    b1vec = par_v[0, :]
    w2vec = par_v[1, :]
    b2s = w2vec[15]
    w2z = jnp.where(lanes < K, w2vec, 0.0)

    def chunk_body(c, _):
        cb = c * C
        cp0 = pltpu.async_copy(uidx_hbm.at[pl.ds(base + cb, C)], uc_v, sem0)
        cp1 = pltpu.async_copy(iidx_hbm.at[pl.ds(base + cb, C)], ic_v, sem1)
        cp0.wait()
        cp1.wait()
        cp0 = pltpu.async_copy(uit_hbm.at[uc_v], un_v, sem0)
        cp1 = pltpu.async_copy(iit_hbm.at[ic_v], in_v, sem1)
        cp0.wait()
        cp1.wait()
        _flatten_rows(un_v, unf_v)
        _flatten_rows(in_v, inf_v)
        copies = []
        for g in range(NG):
            s = pl.ds(g * G, G)
            copies.append(pltpu.async_copy(uc_hbm.at[unf_v.at[g]], ucb_v.at[s], sem0))
            copies.append(pltpu.async_copy(ic_hbm.at[inf_v.at[g]], icb_v.at[s], sem1))
        for cp in copies:
            cp.wait()

        def elem_body(e, _):
            e20 = e * N
            acc = [jnp.zeros((16,), jnp.float32) for _ in range(K)]
            acc = _branch_accum(e20, ucb_v, w1_v, 0, acc)
            acc = _branch_accum(e20, icb_v, w1_v, 1, acc)
            hvec = jnp.zeros((16,), jnp.float32)
            for k in range(K):
                hvec = hvec + jnp.where(lanes == k, jnp.sum(acc[k]), 0.0)
            hvec = jnp.maximum(hvec + b1vec, 0.0)
            z = jnp.sum(hvec * w2z) + b2s
            plsc.store_scatter(hbuf_v, [jnp.full((16,), cb + e, jnp.int32)],
                               jnp.full((16,), z, jnp.float32), mask=lane0)
            return 0

        lax.fori_loop(0, C, elem_body, 0)
        return 0

    lax.fori_loop(0, NCH, chunk_body, 0)

    for g in range(BPW // 16):
        z = hbuf_v[pl.ds(g * 16, 16)]
        z = jnp.maximum(z, 0.0)
        out_v[pl.ds(g * 16, 16)] = 1.0 / (1.0 + jnp.exp(-z))

    pltpu.sync_copy(out_v, out_hbm.at[pl.ds(base, BPW)])


@functools.partial(jax.jit, static_argnames=())
def kernel(user_idxs, item_idxs, user_idx_tensor, item_idx_tensor,
           user_scr_tensor, item_scr_tensor, user_emb_table, item_emb_table,
           W1, b1, W2, b2):
    padcols = jnp.zeros((user_emb_table.shape[0], W - D - N), jnp.float32)
    ucomb = jnp.concatenate([user_emb_table, user_scr_tensor, padcols], axis=1)
    icomb = jnp.concatenate([item_emb_table, item_scr_tensor, padcols], axis=1)
    w1r = W1.reshape(2, N, D, K).transpose(0, 3, 1, 2)
    w1r = jnp.asarray(w1r, jnp.float32)
    par = jnp.zeros((2, 16), jnp.float32)
    par = par.at[0, 0:K].set(b1)
    par = par.at[1, 0:K].set(W2[:, 0])
    par = par.at[1, 15].set(b2[0])

    run = pl.kernel(
        _sc_body,
        out_type=jax.ShapeDtypeStruct((BATCH,), jnp.float32),
        mesh=plsc.VectorSubcoreMesh(core_axis_name="c", subcore_axis_name="s"),
        compiler_params=pltpu.CompilerParams(
            needs_layout_passes=False, use_tc_tiling_on_sc=False),
        scratch_types=[
            pltpu.VMEM((C,), jnp.int32),
            pltpu.VMEM((C,), jnp.int32),
            pltpu.VMEM((C, N), jnp.int32),
            pltpu.VMEM((C, N), jnp.int32),
            pltpu.VMEM((NG, G), jnp.int32),
            pltpu.VMEM((NG, G), jnp.int32),
            pltpu.VMEM((C * N, W), jnp.float32),
            pltpu.VMEM((C * N, W), jnp.float32),
            pltpu.VMEM((2, K, N, D), jnp.float32),
            pltpu.VMEM((2, 16), jnp.float32),
            pltpu.VMEM((BPW,), jnp.float32),
            pltpu.VMEM((BPW,), jnp.float32),
            pltpu.SemaphoreType.DMA,
            pltpu.SemaphoreType.DMA,
            pltpu.SemaphoreType.DMA,
            pltpu.SemaphoreType.DMA,
        ],
    )
    return run(user_idxs, item_idxs, user_idx_tensor, item_idx_tensor,
               ucomb, icomb, w1r, par)
